# baseline (speedup 1.0000x reference)
import functools
import math

import jax
import jax.numpy as jnp
from jax import lax
from jax.experimental import pallas as pl
from jax.experimental.pallas import tpu as pltpu
from jax.experimental.pallas import tpu_sc as plsc

_info = plsc.get_sparse_core_info()
_NC, _NS, _L = _info.num_cores, _info.num_subcores, _info.num_lanes
_NW = _NC * _NS

_CHUNK = 128
_NBUF = 4


@functools.lru_cache(maxsize=None)
def _make_kernel(B, D, scale):
    rows_per_w = B // _NW
    chunks_per_w = rows_per_w // _CHUNK
    assert chunks_per_w % _NBUF == 0

    mesh = plsc.VectorSubcoreMesh(core_axis_name="c", subcore_axis_name="s")

    @functools.partial(
        pl.kernel,
        mesh=mesh,
        out_type=jax.ShapeDtypeStruct((B, D), jnp.float32),
        scratch_types=[
            pltpu.VMEM((chunks_per_w, _CHUNK), jnp.int32),
            pltpu.VMEM((_NBUF, _CHUNK, D), jnp.float32),
            pltpu.VMEM((_NBUF, _CHUNK, D), jnp.float32),
        ]
        + [pltpu.SemaphoreType.DMA] * (2 * _NBUF + 1),
        compiler_params=pltpu.CompilerParams(use_tc_tiling_on_sc=False),
    )
    def k(idx_hbm, table_hbm, out_hbm, idx_v, gbuf, sbuf, *sems):
        isem = sems[0]
        gsems = sems[1 : 1 + _NBUF]
        ssems = sems[1 + _NBUF :]
        wid = lax.axis_index("s") * _NC + lax.axis_index("c")
        base = wid * rows_per_w

        pltpu.async_copy(idx_hbm.at[wid], idx_v, isem).wait()

        for b in range(_NBUF):
            pltpu.async_copy(table_hbm.at[idx_v.at[b]], gbuf.at[b], gsems[b])

        def outer(c0, carry):
            for b in range(_NBUF):
                c = c0 * _NBUF + b
                pltpu.make_async_copy(
                    table_hbm.at[idx_v.at[c]], gbuf.at[b], gsems[b]
                ).wait()

                @pl.when(c0 > 0)
                def _():
                    pltpu.make_async_copy(
                        sbuf.at[b],
                        out_hbm.at[pl.ds(base + (c - _NBUF) * _CHUNK, _CHUNK)],
                        ssems[b],
                    ).wait()

                def scale_body(r, acc):
                    for j in range(D // _L):
                        sbuf[b, r, pl.ds(j * _L, _L)] = (
                            gbuf[b, r, pl.ds(j * _L, _L)] * scale
                        )
                    return acc

                lax.fori_loop(0, _CHUNK, scale_body, 0, unroll=8)

                pltpu.async_copy(
                    sbuf.at[b],
                    out_hbm.at[pl.ds(base + c * _CHUNK, _CHUNK)],
                    ssems[b],
                )

                @pl.when(c + _NBUF < chunks_per_w)
                def _():
                    pltpu.async_copy(
                        table_hbm.at[idx_v.at[c + _NBUF]], gbuf.at[b], gsems[b]
                    )

            return carry

        lax.fori_loop(0, chunks_per_w // _NBUF, outer, 0)

        for b in range(_NBUF):
            c = chunks_per_w - _NBUF + b
            pltpu.make_async_copy(
                sbuf.at[b],
                out_hbm.at[pl.ds(base + c * _CHUNK, _CHUNK)],
                ssems[b],
            ).wait()

    return k


def kernel(x, lut):
    B = x.size
    D = lut.shape[1]
    scale = float(math.sqrt(D))
    idx = x.reshape(_NW, B // (_NW * _CHUNK), _CHUNK).astype(jnp.int32)
    out = _make_kernel(B, D, scale)(idx, lut)
    return out.reshape(x.shape + (D,))
